# Initial kernel scaffold; baseline (speedup 1.0000x reference)
import jax
import jax.numpy as jnp
from jax.experimental import pallas as pl


def kernel(tokens, table, wih0, bias0i, whh_blk, gh_bias, wih1, bias1i, w_lin, b_lin):
    raise NotImplementedError("write your pallas kernel here")


# speedup vs baseline: 1.1061x; 1.1061x over previous
import functools

import jax
import jax.numpy as jnp
from jax import lax
from jax.experimental import pallas as pl
from jax.experimental.pallas import tpu as pltpu


def _slot_rnn_fwd(tok_sm, table_ref, wih0_ref, bias0i_ref, whh_blk_ref,
                  ghb_ref, wih1_ref, bias1i_ref, wlin_ref, blin_ref,
                  o_ref, emb_ref, gi_ref, h_ref):
    i = pl.program_id(0)
    Bs, T, C = o_ref.shape
    V, E = table_ref.shape
    H3 = wih0_ref.shape[1]
    H = H3 // 3

    iota8 = lax.broadcasted_iota(jnp.int32, (8, E), 0)
    for t in range(T):
        for b in range(Bs):
            tok = tok_sm[i * Bs + b, t]
            base = pl.multiple_of((tok >> 3) << 3, 8)
            chunk = table_ref[pl.ds(base, 8), :]
            sel = (iota8 == (tok & 7)).astype(jnp.float32)
            emb_ref[pl.ds(t * Bs + b, 1), :] = jnp.sum(
                chunk * sel, axis=0, keepdims=True)

    gi_ref[...] = (jnp.dot(emb_ref[...], wih0_ref[...],
                           preferred_element_type=jnp.float32)
                   + bias0i_ref[...])

    def gru_steps(w, gb):
        h = None
        for t in range(T):
            gi = gi_ref[pl.ds(t * Bs, Bs), :]
            if h is None:
                gh = jnp.broadcast_to(gb, (Bs, H3))
            else:
                gh = jnp.dot(h, w, preferred_element_type=jnp.float32) + gb
            rz = jax.nn.sigmoid(gi[:, :2 * H] + gh[:, :2 * H])
            n = jnp.tanh(gi[:, 2 * H:] + rz[:, :H] * gh[:, 2 * H:])
            if h is None:
                h = n - rz[:, H:] * n
            else:
                h = n + rz[:, H:] * (h - n)
            h_ref[pl.ds(t * Bs, Bs), :] = h

    gru_steps(whh_blk_ref[0:H, 0:H3], ghb_ref[:, 0:H3])

    gi_ref[...] = (jnp.dot(h_ref[...], wih1_ref[...],
                           preferred_element_type=jnp.float32)
                   + bias1i_ref[...])

    gru_steps(whh_blk_ref[H:2 * H, H3:2 * H3], ghb_ref[:, H3:2 * H3])

    logits = (jnp.dot(h_ref[...], wlin_ref[...],
                      preferred_element_type=jnp.float32) + blin_ref[...])
    rows = [logits[t * Bs:(t + 1) * Bs, :] for t in range(T)]
    m = functools.reduce(jnp.maximum, rows)
    tot = functools.reduce(lambda a, b: a + b,
                           [jnp.exp(r - m) for r in rows])
    lse = m + jnp.log(tot)
    for t in range(T):
        o_ref[:, pl.ds(t, 1), :] = (rows[t] - lse)[:, None, :]


def kernel(tokens, table, wih0, bias0i, whh_blk, gh_bias, wih1, bias1i,
           w_lin, b_lin):
    B, T = tokens.shape
    C = w_lin.shape[1]
    G = 2 if (B % 2 == 0 and (B // 2) % 8 == 0) else 1
    Bs = B // G

    def cs(arr):
        nd = arr.ndim
        return pl.BlockSpec(arr.shape, lambda i, tok, _nd=nd: (0,) * _nd)

    grid_spec = pltpu.PrefetchScalarGridSpec(
        num_scalar_prefetch=1,
        grid=(G,),
        in_specs=[cs(table), cs(wih0), cs(bias0i), cs(whh_blk), cs(gh_bias),
                  cs(wih1), cs(bias1i), cs(w_lin), cs(b_lin)],
        out_specs=pl.BlockSpec((Bs, T, C), lambda i, tok: (i, 0, 0)),
        scratch_shapes=[
            pltpu.VMEM((Bs * T, table.shape[1]), jnp.float32),
            pltpu.VMEM((Bs * T, wih0.shape[1]), jnp.float32),
            pltpu.VMEM((Bs * T, whh_blk.shape[0] // 2), jnp.float32),
        ],
    )
    return pl.pallas_call(
        _slot_rnn_fwd,
        out_shape=jax.ShapeDtypeStruct((B, T, C), jnp.float32),
        grid_spec=grid_spec,
        compiler_params=pltpu.CompilerParams(
            dimension_semantics=("parallel",)),
    )(tokens, table, wih0, bias0i, whh_blk, gh_bias, wih1, bias1i,
      w_lin, b_lin)
